# Initial kernel scaffold; baseline (speedup 1.0000x reference)
import jax
import jax.numpy as jnp
from jax.experimental import pallas as pl


def kernel(x, W, b, gamma, beta):
    raise NotImplementedError("write your pallas kernel here")


# speedup vs baseline: 45.6053x; 45.6053x over previous
import functools

import jax
import jax.numpy as jnp
from jax.experimental import pallas as pl
from jax.experimental.pallas import tpu as pltpu

def _body(x_ref, w_ref, b_ref, g_ref, be_ref, bn_ref, hash_ref):
    N, D = bn_ref.shape
    K = N // 2

    emb = jnp.dot(x_ref[...], w_ref[...].T, preferred_element_type=jnp.float32)
    emb = emb + b_ref[...]
    mean = jnp.mean(emb, axis=0, keepdims=True)
    var = jnp.mean((emb - mean) ** 2, axis=0, keepdims=True)
    bn = (emb - mean) * (jax.lax.rsqrt(var + 1e-5) * g_ref[...]) + be_ref[...]
    bn_ref[...] = bn

    raw = jax.lax.bitcast_convert_type(bn, jnp.int32)
    keys = raw ^ ((raw >> 31) & jnp.int32(0x7FFFFFFF))

    cnt_nonneg = jnp.sum((keys >= 0).astype(jnp.int32), axis=0, keepdims=True)
    T0 = jnp.where(cnt_nonneg >= K, jnp.int32(0), jnp.int32(-2147483648))

    def vbody(i, T):
        cand = T | jnp.left_shift(jnp.int32(1), 30 - i)
        cnt = jnp.sum((keys >= cand).astype(jnp.int32), axis=0, keepdims=True)
        return jnp.where(cnt >= K, cand, T)

    T = jax.lax.fori_loop(0, 31, vbody, T0, unroll=True)

    g = jnp.sum((keys > T).astype(jnp.int32), axis=0, keepdims=True)
    need = K - g
    eq = keys == T
    row = jax.lax.broadcasted_iota(jnp.int32, (N, D), 0)

    def ibody(i, S):
        cand = S | jnp.left_shift(jnp.int32(1), 13 - i)
        f = jnp.sum((eq & (row < cand)).astype(jnp.int32), axis=0, keepdims=True)
        return jnp.where(f < need, cand, S)

    S = jax.lax.fori_loop(0, 14, ibody, jnp.zeros((1, D), jnp.int32), unroll=True)

    plus = (keys > T) | (eq & (row <= S))
    hash_ref[...] = jnp.where(plus, jnp.float32(1.0), jnp.float32(-1.0))


@jax.jit
def kernel(x, W, b, gamma, beta):
    N, G = x.shape
    D = W.shape[0]
    out = pl.pallas_call(
        _body,
        out_shape=(
            jax.ShapeDtypeStruct((N, D), jnp.float32),
            jax.ShapeDtypeStruct((N, D), jnp.float32),
        ),
    )(x, W, b.reshape(1, D), gamma.reshape(1, D), beta.reshape(1, D))
    return out


# speedup vs baseline: 59.6797x; 1.3086x over previous
import jax
import jax.numpy as jnp
from jax.experimental import pallas as pl


def _body(x_ref, w_ref, b_ref, g_ref, be_ref, bn_ref, hash_ref):
    N, D = bn_ref.shape
    H = N // 2
    K = N // 2

    def fold(c):
        return c[:, :D] + c[:, D:]

    def dup(c):
        return jnp.concatenate([c, c], axis=1)

    def cnt(mask):
        return fold(jnp.sum(mask.astype(jnp.int32), axis=0, keepdims=True))

    wt = w_ref[...].T
    top = jnp.dot(x_ref[0:H, :], wt, preferred_element_type=jnp.float32)
    bot = jnp.dot(x_ref[H:N, :], wt, preferred_element_type=jnp.float32)
    embp = jnp.concatenate([top, bot], axis=1) + dup(b_ref[...])

    mean = fold(jnp.sum(embp, axis=0, keepdims=True)) / N
    dev = embp - dup(mean)
    var = fold(jnp.sum(dev * dev, axis=0, keepdims=True)) / N
    scale = jax.lax.rsqrt(var + 1e-5) * g_ref[...]
    bnp = dev * dup(scale) + dup(be_ref[...])
    bn_ref[0:H, :] = bnp[:, 0:D]
    bn_ref[H:N, :] = bnp[:, D:]

    raw = jax.lax.bitcast_convert_type(bnp, jnp.int32)
    keys = raw ^ ((raw >> 31) & jnp.int32(0x7FFFFFFF))

    c0 = cnt(keys >= 0)
    T = jnp.where(c0 >= K, jnp.int32(0), jnp.int32(-2147483648))

    def vbody(i, T):
        cand = T | jnp.left_shift(jnp.int32(1), 30 - i)
        return jnp.where(cnt(keys >= dup(cand)) >= K, cand, T)

    T = jax.lax.fori_loop(0, 31, vbody, T, unroll=True)
    T2 = dup(T)

    gt = keys > T2
    g = cnt(gt)
    need = K - g
    eq = keys == T2
    e = cnt(eq)

    q = jax.lax.broadcasted_iota(jnp.int32, (H, 2 * D), 0)
    lane = jax.lax.broadcasted_iota(jnp.int32, (H, 2 * D), 1)
    row = q + jnp.where(lane >= D, jnp.int32(H), jnp.int32(0))

    def tie_branch():
        def ibody(i, S):
            cand = S | jnp.left_shift(jnp.int32(1), 13 - i)
            f = cnt(eq & (row < dup(cand)))
            return jnp.where(f < need, cand, S)

        return jax.lax.fori_loop(
            0, 14, ibody, jnp.zeros((1, D), jnp.int32), unroll=True)

    def no_tie_branch():
        return jnp.full((1, D), N - 1, jnp.int32)

    S = jax.lax.cond(jnp.max(e) > 1, tie_branch, no_tie_branch)

    plus = gt | (eq & (row <= dup(S)))
    hashp = jnp.where(plus, jnp.float32(1.0), jnp.float32(-1.0))
    hash_ref[0:H, :] = hashp[:, 0:D]
    hash_ref[H:N, :] = hashp[:, D:]


@jax.jit
def kernel(x, W, b, gamma, beta):
    N, G = x.shape
    D = W.shape[0]
    out = pl.pallas_call(
        _body,
        out_shape=(
            jax.ShapeDtypeStruct((N, D), jnp.float32),
            jax.ShapeDtypeStruct((N, D), jnp.float32),
        ),
    )(x, W, b.reshape(1, D), gamma.reshape(1, D), beta.reshape(1, D))
    return out
